# baseline (speedup 1.0000x reference)
import functools

import jax
import jax.numpy as jnp
from jax import lax
from jax.experimental import pallas as pl
from jax.experimental.pallas import tpu as pltpu
from jax.experimental.pallas import tpu_sc as plsc

_N = 10000
_D = 128
_T = 8
_NC = 2
_NS = 16
_NW = _NC * _NS
_L = 16
_K = 128
_NACC = 10240
_RPT = _NACC // _NS


def _sc_segment_sum(ca: int, cb: int, compute_deg: bool):
    cmax = max(ca, cb)
    mesh = plsc.VectorSubcoreMesh(
        core_axis_name="c", subcore_axis_name="s",
        num_cores=_NC, num_subcores=_NS)
    out_type = [jax.ShapeDtypeStruct((_NC, _NACC, _D), jnp.float32)]
    if compute_deg:
        out_type.append(jax.ShapeDtypeStruct((_NC, _NACC), jnp.float32))
    scratch = [
        pltpu.VMEM((cmax, _K), jnp.int32),
        pltpu.VMEM((cmax, _K), jnp.int32),
        pltpu.VMEM((_K, _D), jnp.float32),
        pltpu.VMEM((_K,), jnp.float32),
        pltpu.VMEM_SHARED((_NACC, _D), jnp.float32),
        pltpu.VMEM_SHARED((_NACC,), jnp.float32),
        pltpu.SemaphoreType.DMA,
    ]

    def body(*refs):
        if compute_deg:
            (table, srcs, dsts, out_acc, out_deg,
             src_i, dst_i, rows, ones_v, acc_sh, deg_sh, sem) = refs
        else:
            (table, srcs, dsts, out_acc,
             src_i, dst_i, rows, ones_v, acc_sh, deg_sh, sem) = refs
        c = lax.axis_index("c")
        s = lax.axis_index("s")
        base = s * _RPT

        zero16 = jnp.zeros((_L,), jnp.float32)

        def _zrow(i, carry):
            for jj in range(_D // _L):
                rows[i, pl.ds(jj * _L, _L)] = zero16
            return carry

        lax.fori_loop(0, _K, _zrow, 0)
        for jj in range(_K // _L):
            ones_v[pl.ds(jj * _L, _L)] = jnp.full((_L,), 1.0, jnp.float32)
        for k in range(_RPT // _K):
            pltpu.sync_copy(rows, acc_sh.at[pl.ds(base + k * _K, _K)])
        if compute_deg:
            for k in range(_RPT // _K):
                pltpu.sync_copy(rows.at[0], deg_sh.at[pl.ds(base + k * _K, _K)])
        plsc.subcore_barrier()

        @pl.when(c == 0)
        def _():
            pltpu.sync_copy(srcs.at[pl.ds(s * ca, ca)],
                            src_i.at[pl.ds(0, ca)])
            pltpu.sync_copy(dsts.at[pl.ds(s * ca, ca)],
                            dst_i.at[pl.ds(0, ca)])

        @pl.when(c == 1)
        def _():
            pltpu.sync_copy(srcs.at[pl.ds(_NS * ca + s * cb, cb)],
                            src_i.at[pl.ds(0, cb)])
            pltpu.sync_copy(dsts.at[pl.ds(_NS * ca + s * cb, cb)],
                            dst_i.at[pl.ds(0, cb)])

        def _chunk(j, carry):
            pltpu.async_copy(table.at[src_i.at[j]], rows, sem).wait()
            pltpu.sync_copy(rows, acc_sh.at[dst_i.at[j]], add=True)
            if compute_deg:
                pltpu.sync_copy(ones_v, deg_sh.at[dst_i.at[j]], add=True)
            return carry

        lax.fori_loop(0, jnp.where(c == 0, ca, cb), _chunk, 0)
        plsc.subcore_barrier()

        pltpu.sync_copy(acc_sh.at[pl.ds(base, _RPT)],
                        out_acc.at[c, pl.ds(base, _RPT)])
        if compute_deg:
            pltpu.sync_copy(deg_sh.at[pl.ds(base, _RPT)],
                            out_deg.at[c, pl.ds(base, _RPT)])

    return pl.kernel(body, out_type=tuple(out_type), mesh=mesh,
                     scratch_types=scratch)


def _kan(a, w0, w1, wb):
    e0 = jnp.exp(-((a + 1.0) * 0.5) ** 2)
    e1 = jnp.exp(-((a - 1.0) * 0.5) ** 2)
    sl = a * lax.logistic(a)
    kw = dict(preferred_element_type=jnp.float32, precision=lax.Precision.HIGHEST)
    return jnp.dot(e0, w0, **kw) + jnp.dot(e1, w1, **kw) + jnp.dot(sl, wb, **kw)


def _layernorm(h):
    mu = jnp.mean(h, axis=-1, keepdims=True)
    cent = h - mu
    var = jnp.mean(cent * cent, axis=-1, keepdims=True)
    return cent * lax.rsqrt(var + 1e-5)


def _mean_from_parts(acc_ref, deg_ref):
    d = jnp.maximum(deg_ref[:, 0] + deg_ref[:, 1], 1.0)
    return (acc_ref[0] + acc_ref[1]) / d[:, None]


def _kan_ln_body(acc_ref, deg_ref, w0, w1, wb, o_ref):
    a = _mean_from_parts(acc_ref, deg_ref)
    o_ref[...] = _layernorm(_kan(a, w0[...], w1[...], wb[...]))


def _kan_ln_head_body(acc_ref, deg_ref, w0, w1, wb, h0, h1, hb, o_ref):
    a = _mean_from_parts(acc_ref, deg_ref)
    h = _layernorm(_kan(a, w0[...], w1[...], wb[...]))
    o_ref[...] = _kan(h, h0[...], h1[...], hb[...])


_BLK = 400
_GRID = _N // _BLK


def _tc_specs(n_small):
    full = pl.BlockSpec((_D, _D), lambda i: (0, 0))
    small = pl.BlockSpec((_D, _T), lambda i: (0, 0))
    return ([pl.BlockSpec((_NC, _BLK, _D), lambda i: (0, i, 0)),
             pl.BlockSpec((_BLK, _NC), lambda i: (i, 0))]
            + [full] * 3 + [small] * n_small)


_SPLIT = 0.42


def kernel(x, edge_index, Ws0, Wb0, Ws1, Wb1, Hs, Hb):
    e = edge_index.shape[1]
    etot = e + _N
    need = -(-etot // (_NS * _K))
    ca = max(8, int(round(need * _SPLIT / 8)) * 8)
    cb = -(-(need - ca) // 8) * 8
    cpair = ca + cb
    epad = _NS * cpair * _K

    loop = jnp.arange(_N, dtype=jnp.int32)
    src = jnp.concatenate([
        edge_index[0].astype(jnp.int32), loop,
        jnp.zeros(epad - etot, jnp.int32)]).reshape(_NS * cpair, _K)
    dst = jnp.concatenate([
        edge_index[1].astype(jnp.int32), loop,
        jnp.full(epad - etot, _N, jnp.int32)]).reshape(_NS * cpair, _K)

    w00, w01, wb0 = Ws0[:, 0::2].T, Ws0[:, 1::2].T, Wb0.T
    w10, w11, wb1 = Ws1[:, 0::2].T, Ws1[:, 1::2].T, Wb1.T
    h0, h1, hb = Hs[:, 0::2].T, Hs[:, 1::2].T, Hb.T

    acc1, deg = _sc_segment_sum(ca, cb, True)(x, src, dst)
    deg_t = deg.T

    h = pl.pallas_call(
        _kan_ln_body,
        grid=(_GRID,),
        in_specs=_tc_specs(0),
        out_specs=pl.BlockSpec((_BLK, _D), lambda i: (i, 0)),
        out_shape=jax.ShapeDtypeStruct((_N, _D), jnp.float32),
    )(acc1, deg_t, w00, w01, wb0)

    (acc2,) = _sc_segment_sum(ca, cb, False)(h, src, dst)

    out = pl.pallas_call(
        _kan_ln_head_body,
        grid=(_GRID,),
        in_specs=_tc_specs(3),
        out_specs=pl.BlockSpec((_BLK, _T), lambda i: (i, 0)),
        out_shape=jax.ShapeDtypeStruct((_N, _T), jnp.float32),
    )(acc2, deg_t, w10, w11, wb1, h0, h1, hb)
    return out


# speedup vs baseline: 1.0003x; 1.0003x over previous
import functools

import jax
import jax.numpy as jnp
from jax import lax
from jax.experimental import pallas as pl
from jax.experimental.pallas import tpu as pltpu
from jax.experimental.pallas import tpu_sc as plsc

_N = 10000
_D = 128
_T = 8
_NC = 2
_NS = 16
_NW = _NC * _NS
_L = 16
_K = 128
_NACC = 10240
_RPT = _NACC // _NS


def _sc_segment_sum(ca: int, cb: int, compute_deg: bool):
    cmax = max(ca, cb)
    mesh = plsc.VectorSubcoreMesh(
        core_axis_name="c", subcore_axis_name="s",
        num_cores=_NC, num_subcores=_NS)
    out_type = [jax.ShapeDtypeStruct((_NC, _NACC, _D), jnp.float32)]
    if compute_deg:
        out_type.append(jax.ShapeDtypeStruct((_NC, _NACC), jnp.float32))
    scratch = [
        pltpu.VMEM((cmax, _K), jnp.int32),
        pltpu.VMEM((cmax, _K), jnp.int32),
        pltpu.VMEM((_K, _D), jnp.float32),
        pltpu.VMEM((_K,), jnp.float32),
        pltpu.VMEM_SHARED((_NACC, _D), jnp.float32),
        pltpu.VMEM_SHARED((_NACC,), jnp.float32),
        pltpu.SemaphoreType.DMA,
    ]

    def body(*refs):
        if compute_deg:
            (table, srcs, dsts, out_acc, out_deg,
             src_i, dst_i, rows, ones_v, acc_sh, deg_sh, sem) = refs
        else:
            (table, srcs, dsts, out_acc,
             src_i, dst_i, rows, ones_v, acc_sh, deg_sh, sem) = refs
        c = lax.axis_index("c")
        s = lax.axis_index("s")
        base = s * _RPT

        zero16 = jnp.zeros((_L,), jnp.float32)

        def _zrow(i, carry):
            for jj in range(_D // _L):
                rows[i, pl.ds(jj * _L, _L)] = zero16
            return carry

        lax.fori_loop(0, _K, _zrow, 0)
        for jj in range(_K // _L):
            ones_v[pl.ds(jj * _L, _L)] = jnp.full((_L,), 1.0, jnp.float32)
        for k in range(_RPT // _K):
            pltpu.sync_copy(rows, acc_sh.at[pl.ds(base + k * _K, _K)])
        if compute_deg:
            for k in range(_RPT // _K):
                pltpu.sync_copy(rows.at[0], deg_sh.at[pl.ds(base + k * _K, _K)])
        plsc.subcore_barrier()

        @pl.when(c == 0)
        def _():
            pltpu.sync_copy(srcs.at[pl.ds(s * ca, ca)],
                            src_i.at[pl.ds(0, ca)])
            pltpu.sync_copy(dsts.at[pl.ds(s * ca, ca)],
                            dst_i.at[pl.ds(0, ca)])

        @pl.when(c == 1)
        def _():
            pltpu.sync_copy(srcs.at[pl.ds(_NS * ca + s * cb, cb)],
                            src_i.at[pl.ds(0, cb)])
            pltpu.sync_copy(dsts.at[pl.ds(_NS * ca + s * cb, cb)],
                            dst_i.at[pl.ds(0, cb)])

        def _chunk(j, carry):
            pltpu.async_copy(table.at[src_i.at[j]], rows, sem).wait()
            pltpu.sync_copy(rows, acc_sh.at[dst_i.at[j]], add=True)
            if compute_deg:
                pltpu.sync_copy(ones_v, deg_sh.at[dst_i.at[j]], add=True)
            return carry

        @pl.when(c == 0)
        def _():
            lax.fori_loop(0, ca, _chunk, 0)

        @pl.when(c == 1)
        def _():
            lax.fori_loop(0, cb, _chunk, 0)

        plsc.subcore_barrier()

        pltpu.sync_copy(acc_sh.at[pl.ds(base, _RPT)],
                        out_acc.at[c, pl.ds(base, _RPT)])
        if compute_deg:
            pltpu.sync_copy(deg_sh.at[pl.ds(base, _RPT)],
                            out_deg.at[c, pl.ds(base, _RPT)])

    return pl.kernel(body, out_type=tuple(out_type), mesh=mesh,
                     scratch_types=scratch)


def _kan(a, w0, w1, wb):
    e0 = jnp.exp(-((a + 1.0) * 0.5) ** 2)
    e1 = jnp.exp(-((a - 1.0) * 0.5) ** 2)
    sl = a * lax.logistic(a)
    kw = dict(preferred_element_type=jnp.float32, precision=lax.Precision.HIGHEST)
    return jnp.dot(e0, w0, **kw) + jnp.dot(e1, w1, **kw) + jnp.dot(sl, wb, **kw)


def _layernorm(h):
    mu = jnp.mean(h, axis=-1, keepdims=True)
    cent = h - mu
    var = jnp.mean(cent * cent, axis=-1, keepdims=True)
    return cent * lax.rsqrt(var + 1e-5)


def _mean_from_parts(acc_ref, deg_ref):
    d = jnp.maximum(deg_ref[:, 0] + deg_ref[:, 1], 1.0)
    return (acc_ref[0] + acc_ref[1]) / d[:, None]


def _kan_ln_body(acc_ref, deg_ref, w0, w1, wb, o_ref):
    a = _mean_from_parts(acc_ref, deg_ref)
    o_ref[...] = _layernorm(_kan(a, w0[...], w1[...], wb[...]))


def _kan_ln_head_body(acc_ref, deg_ref, w0, w1, wb, h0, h1, hb, o_ref):
    a = _mean_from_parts(acc_ref, deg_ref)
    h = _layernorm(_kan(a, w0[...], w1[...], wb[...]))
    o_ref[...] = _kan(h, h0[...], h1[...], hb[...])


_BLK = 400
_GRID = _N // _BLK


def _tc_specs(n_small):
    full = pl.BlockSpec((_D, _D), lambda i: (0, 0))
    small = pl.BlockSpec((_D, _T), lambda i: (0, 0))
    return ([pl.BlockSpec((_NC, _BLK, _D), lambda i: (0, i, 0)),
             pl.BlockSpec((_BLK, _NC), lambda i: (i, 0))]
            + [full] * 3 + [small] * n_small)


_SPLIT = 0.42


def kernel(x, edge_index, Ws0, Wb0, Ws1, Wb1, Hs, Hb):
    e = edge_index.shape[1]
    etot = e + _N
    need = -(-etot // (_NS * _K))
    ca = max(8, int(round(need * _SPLIT / 8)) * 8)
    cb = -(-(need - ca) // 8) * 8
    cpair = ca + cb
    epad = _NS * cpair * _K

    loop = jnp.arange(_N, dtype=jnp.int32)
    src = jnp.concatenate([
        edge_index[0].astype(jnp.int32), loop,
        jnp.zeros(epad - etot, jnp.int32)]).reshape(_NS * cpair, _K)
    dst = jnp.concatenate([
        edge_index[1].astype(jnp.int32), loop,
        jnp.full(epad - etot, _N, jnp.int32)]).reshape(_NS * cpair, _K)

    w00, w01, wb0 = Ws0[:, 0::2].T, Ws0[:, 1::2].T, Wb0.T
    w10, w11, wb1 = Ws1[:, 0::2].T, Ws1[:, 1::2].T, Wb1.T
    h0, h1, hb = Hs[:, 0::2].T, Hs[:, 1::2].T, Hb.T

    acc1, deg = _sc_segment_sum(ca, cb, True)(x, src, dst)
    deg_t = deg.T

    h = pl.pallas_call(
        _kan_ln_body,
        grid=(_GRID,),
        in_specs=_tc_specs(0),
        out_specs=pl.BlockSpec((_BLK, _D), lambda i: (i, 0)),
        out_shape=jax.ShapeDtypeStruct((_N, _D), jnp.float32),
    )(acc1, deg_t, w00, w01, wb0)

    (acc2,) = _sc_segment_sum(ca, cb, False)(h, src, dst)

    out = pl.pallas_call(
        _kan_ln_head_body,
        grid=(_GRID,),
        in_specs=_tc_specs(3),
        out_specs=pl.BlockSpec((_BLK, _T), lambda i: (i, 0)),
        out_shape=jax.ShapeDtypeStruct((_N, _T), jnp.float32),
    )(acc2, deg_t, w10, w11, wb1, h0, h1, hb)
    return out


# speedup vs baseline: 1.0205x; 1.0202x over previous
import functools

import jax
import jax.numpy as jnp
from jax import lax
from jax.experimental import pallas as pl
from jax.experimental.pallas import tpu as pltpu
from jax.experimental.pallas import tpu_sc as plsc

_N = 10000
_D = 128
_T = 8
_NC = 2
_NS = 16
_NW = _NC * _NS
_L = 16
_K = 128
_NACC = 10240
_RPT = _NACC // _NS


def _sc_segment_sum(ca: int, cb: int, compute_deg: bool):
    cmax = max(ca, cb)
    mesh = plsc.VectorSubcoreMesh(
        core_axis_name="c", subcore_axis_name="s",
        num_cores=_NC, num_subcores=_NS)
    out_type = [jax.ShapeDtypeStruct((_NC, _NACC, _D), jnp.float32)]
    if compute_deg:
        out_type.append(jax.ShapeDtypeStruct((_NC, _NACC), jnp.float32))
    scratch = [
        pltpu.VMEM((cmax, _K), jnp.int32),
        pltpu.VMEM((cmax, _K), jnp.int32),
        pltpu.VMEM((_K, _D), jnp.float32),
        pltpu.VMEM((_K,), jnp.float32),
        pltpu.VMEM_SHARED((_NACC, _D), jnp.float32),
        pltpu.VMEM_SHARED((_NACC,), jnp.float32),
        pltpu.SemaphoreType.DMA,
    ]

    def body(*refs):
        if compute_deg:
            (table, srcs, dsts, out_acc, out_deg,
             src_i, dst_i, rows, ones_v, acc_sh, deg_sh, sem) = refs
        else:
            (table, srcs, dsts, out_acc,
             src_i, dst_i, rows, ones_v, acc_sh, deg_sh, sem) = refs
        c = lax.axis_index("c")
        s = lax.axis_index("s")
        base = s * _RPT

        zero16 = jnp.zeros((_L,), jnp.float32)

        def _zrow(i, carry):
            for jj in range(_D // _L):
                rows[i, pl.ds(jj * _L, _L)] = zero16
            return carry

        lax.fori_loop(0, _K, _zrow, 0)
        for jj in range(_K // _L):
            ones_v[pl.ds(jj * _L, _L)] = jnp.full((_L,), 1.0, jnp.float32)
        for k in range(_RPT // _K):
            pltpu.sync_copy(rows, acc_sh.at[pl.ds(base + k * _K, _K)])
        if compute_deg:
            for k in range(_RPT // _K):
                pltpu.sync_copy(rows.at[0], deg_sh.at[pl.ds(base + k * _K, _K)])
        plsc.subcore_barrier()

        @pl.when(c == 0)
        def _():
            pltpu.sync_copy(srcs.at[pl.ds(s * ca, ca)],
                            src_i.at[pl.ds(0, ca)])
            pltpu.sync_copy(dsts.at[pl.ds(s * ca, ca)],
                            dst_i.at[pl.ds(0, ca)])

        @pl.when(c == 1)
        def _():
            pltpu.sync_copy(srcs.at[pl.ds(_NS * ca + s * cb, cb)],
                            src_i.at[pl.ds(0, cb)])
            pltpu.sync_copy(dsts.at[pl.ds(_NS * ca + s * cb, cb)],
                            dst_i.at[pl.ds(0, cb)])

        def _chunk(j, carry):
            pltpu.async_copy(table.at[src_i.at[j]], rows, sem).wait()
            pltpu.sync_copy(rows, acc_sh.at[dst_i.at[j]], add=True)
            if compute_deg:
                pltpu.sync_copy(ones_v, deg_sh.at[dst_i.at[j]], add=True)
            return carry

        @pl.when(c == 0)
        def _():
            lax.fori_loop(0, ca, _chunk, 0)

        @pl.when(c == 1)
        def _():
            lax.fori_loop(0, cb, _chunk, 0)

        plsc.subcore_barrier()

        pltpu.sync_copy(acc_sh.at[pl.ds(base, _RPT)],
                        out_acc.at[c, pl.ds(base, _RPT)])
        if compute_deg:
            pltpu.sync_copy(deg_sh.at[pl.ds(base, _RPT)],
                            out_deg.at[c, pl.ds(base, _RPT)])

    return pl.kernel(body, out_type=tuple(out_type), mesh=mesh,
                     scratch_types=scratch)


def _kan(a, w0, w1, wb):
    e0 = jnp.exp(-((a + 1.0) * 0.5) ** 2)
    e1 = jnp.exp(-((a - 1.0) * 0.5) ** 2)
    sl = a * lax.logistic(a)
    kw = dict(preferred_element_type=jnp.float32, precision=lax.Precision.HIGHEST)
    return jnp.dot(e0, w0, **kw) + jnp.dot(e1, w1, **kw) + jnp.dot(sl, wb, **kw)


def _layernorm(h):
    mu = jnp.mean(h, axis=-1, keepdims=True)
    cent = h - mu
    var = jnp.mean(cent * cent, axis=-1, keepdims=True)
    return cent * lax.rsqrt(var + 1e-5)


def _mean_from_parts(acc_ref, deg_ref):
    d = jnp.maximum(deg_ref[:, 0] + deg_ref[:, 1], 1.0)
    return (acc_ref[0] + acc_ref[1]) / d[:, None]


def _kan_ln_body(acc_ref, deg_ref, w0, w1, wb, o_ref):
    a = _mean_from_parts(acc_ref, deg_ref)
    o_ref[...] = _layernorm(_kan(a, w0[...], w1[...], wb[...]))


def _kan_ln_head_body(acc_ref, deg_ref, w0, w1, wb, h0, h1, hb, o_ref):
    a = _mean_from_parts(acc_ref, deg_ref)
    h = _layernorm(_kan(a, w0[...], w1[...], wb[...]))
    o_ref[...] = _kan(h, h0[...], h1[...], hb[...])


_BLK = 400
_GRID = _N // _BLK


def _tc_specs(n_small):
    full = pl.BlockSpec((_D, _D), lambda i: (0, 0))
    small = pl.BlockSpec((_D, _T), lambda i: (0, 0))
    return ([pl.BlockSpec((_NC, _BLK, _D), lambda i: (0, i, 0)),
             pl.BlockSpec((_BLK, _NC), lambda i: (i, 0))]
            + [full] * 3 + [small] * n_small)


_SPLIT = 0.5


def kernel(x, edge_index, Ws0, Wb0, Ws1, Wb1, Hs, Hb):
    e = edge_index.shape[1]
    etot = e + _N
    need = -(-etot // (_NS * _K))
    ca = max(8, int(round(need * _SPLIT / 8)) * 8)
    cb = -(-(need - ca) // 8) * 8
    cpair = ca + cb
    epad = _NS * cpair * _K

    loop = jnp.arange(_N, dtype=jnp.int32)
    src = jnp.concatenate([
        edge_index[0].astype(jnp.int32), loop,
        jnp.zeros(epad - etot, jnp.int32)]).reshape(_NS * cpair, _K)
    dst = jnp.concatenate([
        edge_index[1].astype(jnp.int32), loop,
        jnp.full(epad - etot, _N, jnp.int32)]).reshape(_NS * cpair, _K)

    w00, w01, wb0 = Ws0[:, 0::2].T, Ws0[:, 1::2].T, Wb0.T
    w10, w11, wb1 = Ws1[:, 0::2].T, Ws1[:, 1::2].T, Wb1.T
    h0, h1, hb = Hs[:, 0::2].T, Hs[:, 1::2].T, Hb.T

    acc1, deg = _sc_segment_sum(ca, cb, True)(x, src, dst)
    deg_t = deg.T

    h = pl.pallas_call(
        _kan_ln_body,
        grid=(_GRID,),
        in_specs=_tc_specs(0),
        out_specs=pl.BlockSpec((_BLK, _D), lambda i: (i, 0)),
        out_shape=jax.ShapeDtypeStruct((_N, _D), jnp.float32),
    )(acc1, deg_t, w00, w01, wb0)

    (acc2,) = _sc_segment_sum(ca, cb, False)(h, src, dst)

    out = pl.pallas_call(
        _kan_ln_head_body,
        grid=(_GRID,),
        in_specs=_tc_specs(3),
        out_specs=pl.BlockSpec((_BLK, _T), lambda i: (i, 0)),
        out_shape=jax.ShapeDtypeStruct((_N, _T), jnp.float32),
    )(acc2, deg_t, w10, w11, wb1, h0, h1, hb)
    return out


# speedup vs baseline: 2.9811x; 2.9213x over previous
import functools

import jax
import jax.numpy as jnp
from jax import lax
from jax.experimental import pallas as pl
from jax.experimental.pallas import tpu as pltpu
from jax.experimental.pallas import tpu_sc as plsc

_N = 10000
_D = 128
_T = 8
_NC = 2
_NS = 16
_NW = _NC * _NS
_L = 16
_K = 128
_NACC = 10240
_RPT = _NACC // _NS


def _sc_segment_sum(chunks: int, compute_deg: bool):
    mesh = plsc.VectorSubcoreMesh(
        core_axis_name="c", subcore_axis_name="s",
        num_cores=_NC, num_subcores=_NS)
    out_type = [jax.ShapeDtypeStruct((_NC, _NACC, _D), jnp.float32)]
    if compute_deg:
        out_type.append(jax.ShapeDtypeStruct((_NC, _NACC), jnp.float32))
    scratch = [
        pltpu.VMEM((chunks, _K), jnp.int32),
        pltpu.VMEM((chunks, _K), jnp.int32),
        pltpu.VMEM((_K, _D), jnp.float32),
        pltpu.VMEM((_K,), jnp.float32),
        pltpu.VMEM_SHARED((_NACC, _D), jnp.float32),
        pltpu.VMEM_SHARED((_NACC,), jnp.float32),
        pltpu.SemaphoreType.DMA,
    ]

    def body(*refs):
        if compute_deg:
            (table, srcs, dsts, out_acc, out_deg,
             src_i, dst_i, rows, ones_v, acc_sh, deg_sh, sem) = refs
        else:
            (table, srcs, dsts, out_acc,
             src_i, dst_i, rows, ones_v, acc_sh, deg_sh, sem) = refs
        c = lax.axis_index("c")
        s = lax.axis_index("s")
        wid = s * _NC + c
        base = s * _RPT

        zero16 = jnp.zeros((_L,), jnp.float32)

        def _zrow(i, carry):
            for jj in range(_D // _L):
                rows[i, pl.ds(jj * _L, _L)] = zero16
            return carry

        lax.fori_loop(0, _K, _zrow, 0)
        for jj in range(_K // _L):
            ones_v[pl.ds(jj * _L, _L)] = jnp.full((_L,), 1.0, jnp.float32)
        for k in range(_RPT // _K):
            pltpu.sync_copy(rows, acc_sh.at[pl.ds(base + k * _K, _K)])
        if compute_deg:
            for k in range(_RPT // _K):
                pltpu.sync_copy(rows.at[0], deg_sh.at[pl.ds(base + k * _K, _K)])
        plsc.subcore_barrier()

        pltpu.sync_copy(srcs.at[wid], src_i)
        pltpu.sync_copy(dsts.at[wid], dst_i)

        def _chunk(j, carry):
            pltpu.async_copy(table.at[src_i.at[j]], rows, sem).wait()
            pltpu.sync_copy(rows, acc_sh.at[dst_i.at[j]], add=True)
            if compute_deg:
                pltpu.sync_copy(ones_v, deg_sh.at[dst_i.at[j]], add=True)
            return carry

        lax.fori_loop(0, chunks, _chunk, 0)
        plsc.subcore_barrier()

        pltpu.sync_copy(acc_sh.at[pl.ds(base, _RPT)],
                        out_acc.at[c, pl.ds(base, _RPT)])
        if compute_deg:
            pltpu.sync_copy(deg_sh.at[pl.ds(base, _RPT)],
                            out_deg.at[c, pl.ds(base, _RPT)])

    return pl.kernel(body, out_type=tuple(out_type), mesh=mesh,
                     scratch_types=scratch)


def _kan(a, w0, w1, wb):
    e0 = jnp.exp(-((a + 1.0) * 0.5) ** 2)
    e1 = jnp.exp(-((a - 1.0) * 0.5) ** 2)
    sl = a * lax.logistic(a)
    kw = dict(preferred_element_type=jnp.float32, precision=lax.Precision.HIGHEST)
    return jnp.dot(e0, w0, **kw) + jnp.dot(e1, w1, **kw) + jnp.dot(sl, wb, **kw)


def _layernorm(h):
    mu = jnp.mean(h, axis=-1, keepdims=True)
    cent = h - mu
    var = jnp.mean(cent * cent, axis=-1, keepdims=True)
    return cent * lax.rsqrt(var + 1e-5)


def _mean_from_parts(acc_ref, deg_ref):
    d = jnp.maximum(deg_ref[:, 0] + deg_ref[:, 1], 1.0)
    return (acc_ref[0] + acc_ref[1]) / d[:, None]


def _kan_ln_body(acc_ref, deg_ref, w0, w1, wb, o_ref):
    a = _mean_from_parts(acc_ref, deg_ref)
    o_ref[...] = _layernorm(_kan(a, w0[...], w1[...], wb[...]))


def _kan_ln_head_body(acc_ref, deg_ref, w0, w1, wb, h0, h1, hb, o_ref):
    a = _mean_from_parts(acc_ref, deg_ref)
    h = _layernorm(_kan(a, w0[...], w1[...], wb[...]))
    o_ref[...] = _kan(h, h0[...], h1[...], hb[...])


_BLK = 400
_GRID = _N // _BLK


def _tc_specs(n_small):
    full = pl.BlockSpec((_D, _D), lambda i: (0, 0))
    small = pl.BlockSpec((_D, _T), lambda i: (0, 0))
    return ([pl.BlockSpec((_NC, _BLK, _D), lambda i: (0, i, 0)),
             pl.BlockSpec((_BLK, _NC), lambda i: (i, 0))]
            + [full] * 3 + [small] * n_small)


def kernel(x, edge_index, Ws0, Wb0, Ws1, Wb1, Hs, Hb):
    e = edge_index.shape[1]
    etot = e + _N
    chunks = -(-etot // (_NW * _K))
    epad = _NW * chunks * _K

    loop = jnp.arange(_N, dtype=jnp.int32)
    npad = epad - etot
    pad_dst = _N + jnp.arange(npad, dtype=jnp.int32) % (_NACC - _N)
    src = jnp.concatenate([
        edge_index[0].astype(jnp.int32), loop,
        jnp.zeros(npad, jnp.int32)]).reshape(_NW, chunks, _K)
    dst = jnp.concatenate([
        edge_index[1].astype(jnp.int32), loop,
        pad_dst]).reshape(_NW, chunks, _K)

    w00, w01, wb0 = Ws0[:, 0::2].T, Ws0[:, 1::2].T, Wb0.T
    w10, w11, wb1 = Ws1[:, 0::2].T, Ws1[:, 1::2].T, Wb1.T
    h0, h1, hb = Hs[:, 0::2].T, Hs[:, 1::2].T, Hb.T

    acc1, deg = _sc_segment_sum(chunks, True)(x, src, dst)
    deg_t = deg.T

    h = pl.pallas_call(
        _kan_ln_body,
        grid=(_GRID,),
        in_specs=_tc_specs(0),
        out_specs=pl.BlockSpec((_BLK, _D), lambda i: (i, 0)),
        out_shape=jax.ShapeDtypeStruct((_N, _D), jnp.float32),
    )(acc1, deg_t, w00, w01, wb0)

    (acc2,) = _sc_segment_sum(chunks, False)(h, src, dst)

    out = pl.pallas_call(
        _kan_ln_head_body,
        grid=(_GRID,),
        in_specs=_tc_specs(3),
        out_specs=pl.BlockSpec((_BLK, _T), lambda i: (i, 0)),
        out_shape=jax.ShapeDtypeStruct((_N, _T), jnp.float32),
    )(acc2, deg_t, w10, w11, wb1, h0, h1, hb)
    return out


# speedup vs baseline: 3.2227x; 1.0810x over previous
import functools

import jax
import jax.numpy as jnp
from jax import lax
from jax.experimental import pallas as pl
from jax.experimental.pallas import tpu as pltpu
from jax.experimental.pallas import tpu_sc as plsc

_N = 10000
_D = 128
_T = 8
_NC = 2
_NS = 16
_NW = _NC * _NS
_L = 16
_K = 128
_DH = _D // 2
_NACC = 10240
_RPT = _NACC // _NS


def _sc_segment_sum(chunks: int, compute_deg: bool):
    assert chunks % 2 == 0 and chunks >= 4
    mesh = plsc.VectorSubcoreMesh(
        core_axis_name="c", subcore_axis_name="s",
        num_cores=_NC, num_subcores=_NS)
    out_type = [jax.ShapeDtypeStruct((_NC, _NACC, _DH), jnp.float32)]
    if compute_deg:
        out_type.append(jax.ShapeDtypeStruct((_NC, _NACC), jnp.float32))
    scratch = [
        pltpu.VMEM((chunks, _K), jnp.int32),
        pltpu.VMEM((chunks, _K), jnp.int32),
        [pltpu.VMEM((_K, _DH), jnp.float32) for _ in range(2)],
        pltpu.VMEM((_K,), jnp.float32),
        pltpu.VMEM_SHARED((_NACC, _DH), jnp.float32),
        pltpu.VMEM_SHARED((_NACC,), jnp.float32),
        [pltpu.SemaphoreType.DMA for _ in range(2)],
        [pltpu.SemaphoreType.DMA for _ in range(2)],
    ]

    def body(*refs):
        if compute_deg:
            (table, srcs, dsts, out_acc, out_deg,
             src_i, dst_i, rows, ones_v, acc_sh, deg_sh, sem_g, sem_s) = refs
        else:
            (table, srcs, dsts, out_acc,
             src_i, dst_i, rows, ones_v, acc_sh, deg_sh, sem_g, sem_s) = refs
        c = lax.axis_index("c")
        s = lax.axis_index("s")
        base = s * _RPT

        zero16 = jnp.zeros((_L,), jnp.float32)

        def _zrow(i, carry):
            for jj in range(_DH // _L):
                rows[0][i, pl.ds(jj * _L, _L)] = zero16
            return carry

        lax.fori_loop(0, _K, _zrow, 0)
        for jj in range(_K // _L):
            ones_v[pl.ds(jj * _L, _L)] = jnp.full((_L,), 1.0, jnp.float32)
        for k in range(_RPT // _K):
            pltpu.sync_copy(rows[0], acc_sh.at[pl.ds(base + k * _K, _K)])
        if compute_deg:
            for k in range(_RPT // _DH):
                pltpu.sync_copy(rows[0].at[0],
                                deg_sh.at[pl.ds(base + k * _DH, _DH)])
        plsc.subcore_barrier()

        pltpu.sync_copy(srcs.at[c, s], src_i)
        pltpu.sync_copy(dsts.at[s], dst_i)

        def _retire(j, b):
            pltpu.make_async_copy(rows[b], acc_sh.at[dst_i.at[j]],
                                  sem_s[b]).wait()
            if compute_deg:
                pltpu.make_async_copy(ones_v, deg_sh.at[dst_i.at[j]],
                                      sem_s[b]).wait()

        def _slot(j, b, first=False, last=False):
            pltpu.make_async_copy(table.at[src_i.at[j]], rows[b],
                                  sem_g[b]).wait()
            pltpu.async_copy(rows[b], acc_sh.at[dst_i.at[j]], sem_s[b],
                             add=True)
            if compute_deg:
                pltpu.async_copy(ones_v, deg_sh.at[dst_i.at[j]], sem_s[b],
                                 add=True)
            if not first:
                _retire(j - 1, 1 - b)
            if not last:
                pltpu.async_copy(table.at[src_i.at[j + 1]], rows[1 - b],
                                 sem_g[1 - b])

        pltpu.async_copy(table.at[src_i.at[0]], rows[0], sem_g[0])
        _slot(0, 0, first=True)
        _slot(1, 1)

        def _pair(p, carry):
            _slot(2 * p, 0)
            _slot(2 * p + 1, 1)
            return carry

        lax.fori_loop(1, chunks // 2 - 1, _pair, 0)
        _slot(chunks - 2, 0)
        _slot(chunks - 1, 1, last=True)
        _retire(chunks - 1, 1)
        plsc.subcore_barrier()

        pltpu.sync_copy(acc_sh.at[pl.ds(base, _RPT)],
                        out_acc.at[c, pl.ds(base, _RPT)])
        if compute_deg:
            pltpu.sync_copy(deg_sh.at[pl.ds(base, _RPT)],
                            out_deg.at[c, pl.ds(base, _RPT)])

    return pl.kernel(
        body, out_type=tuple(out_type), mesh=mesh, scratch_types=scratch,
        compiler_params=pltpu.CompilerParams(use_tc_tiling_on_sc=False))


def _kan(a, w0, w1, wb):
    e0 = jnp.exp(-((a + 1.0) * 0.5) ** 2)
    e1 = jnp.exp(-((a - 1.0) * 0.5) ** 2)
    sl = a * lax.logistic(a)
    kw = dict(preferred_element_type=jnp.float32, precision=lax.Precision.HIGHEST)
    return jnp.dot(e0, w0, **kw) + jnp.dot(e1, w1, **kw) + jnp.dot(sl, wb, **kw)


def _layernorm(h):
    mu = jnp.mean(h, axis=-1, keepdims=True)
    cent = h - mu
    var = jnp.mean(cent * cent, axis=-1, keepdims=True)
    return cent * lax.rsqrt(var + 1e-5)


def _mean_from_parts(acc_ref, deg_ref):
    d = jnp.maximum(deg_ref[:, 0], 1.0)
    a = jnp.concatenate([acc_ref[0], acc_ref[1]], axis=-1)
    return a / d[:, None]


def _kan_ln_body(acc_ref, deg_ref, w0, w1, wb, o_ref):
    a = _mean_from_parts(acc_ref, deg_ref)
    o_ref[...] = _layernorm(_kan(a, w0[...], w1[...], wb[...]))


def _kan_ln_head_body(acc_ref, deg_ref, w0, w1, wb, h0, h1, hb, o_ref):
    a = _mean_from_parts(acc_ref, deg_ref)
    h = _layernorm(_kan(a, w0[...], w1[...], wb[...]))
    o_ref[...] = _kan(h, h0[...], h1[...], hb[...])


_BLK = 400
_GRID = _N // _BLK


def _tc_specs(n_small):
    full = pl.BlockSpec((_D, _D), lambda i: (0, 0))
    small = pl.BlockSpec((_D, _T), lambda i: (0, 0))
    return ([pl.BlockSpec((_NC, _BLK, _DH), lambda i: (0, i, 0)),
             pl.BlockSpec((_BLK, _NC), lambda i: (i, 0))]
            + [full] * 3 + [small] * n_small)


def _stack_halves(t):
    return jnp.concatenate([t[:, :_DH], t[:, _DH:]], axis=0)


def kernel(x, edge_index, Ws0, Wb0, Ws1, Wb1, Hs, Hb):
    e = edge_index.shape[1]
    etot = e + _N
    chunks = -(-etot // (_NS * _K))
    chunks += chunks % 2
    epad = _NS * chunks * _K

    loop = jnp.arange(_N, dtype=jnp.int32)
    npad = epad - etot
    pad_dst = _N + jnp.arange(npad, dtype=jnp.int32) % (_NACC - _N)
    src = jnp.concatenate([
        edge_index[0].astype(jnp.int32), loop,
        jnp.zeros(npad, jnp.int32)]).reshape(_NS, chunks, _K)
    src = jnp.stack([src, src + _N])
    dst = jnp.concatenate([
        edge_index[1].astype(jnp.int32), loop,
        pad_dst]).reshape(_NS, chunks, _K)

    w00, w01, wb0 = Ws0[:, 0::2].T, Ws0[:, 1::2].T, Wb0.T
    w10, w11, wb1 = Ws1[:, 0::2].T, Ws1[:, 1::2].T, Wb1.T
    h0, h1, hb = Hs[:, 0::2].T, Hs[:, 1::2].T, Hb.T

    acc1, deg = _sc_segment_sum(chunks, True)(_stack_halves(x), src, dst)
    deg_t = deg.T

    h = pl.pallas_call(
        _kan_ln_body,
        grid=(_GRID,),
        in_specs=_tc_specs(0),
        out_specs=pl.BlockSpec((_BLK, _D), lambda i: (i, 0)),
        out_shape=jax.ShapeDtypeStruct((_N, _D), jnp.float32),
    )(acc1, deg_t, w00, w01, wb0)

    (acc2,) = _sc_segment_sum(chunks, False)(_stack_halves(h), src, dst)

    out = pl.pallas_call(
        _kan_ln_head_body,
        grid=(_GRID,),
        in_specs=_tc_specs(3),
        out_specs=pl.BlockSpec((_BLK, _T), lambda i: (i, 0)),
        out_shape=jax.ShapeDtypeStruct((_N, _T), jnp.float32),
    )(acc2, deg_t, w10, w11, wb1, h0, h1, hb)
    return out


# speedup vs baseline: 3.2298x; 1.0022x over previous
import functools

import jax
import jax.numpy as jnp
from jax import lax
from jax.experimental import pallas as pl
from jax.experimental.pallas import tpu as pltpu
from jax.experimental.pallas import tpu_sc as plsc

_N = 10000
_D = 128
_T = 8
_NC = 2
_NS = 16
_NW = _NC * _NS
_L = 16
_K = 128
_DH = _D // 2
_NACC = 10240
_RPT = _NACC // _NS


def _sc_segment_sum(chunks: int, compute_deg: bool):
    assert chunks % 2 == 0 and chunks >= 4
    mesh = plsc.VectorSubcoreMesh(
        core_axis_name="c", subcore_axis_name="s",
        num_cores=_NC, num_subcores=_NS)
    out_type = [jax.ShapeDtypeStruct((_NC, _NACC, _DH), jnp.float32)]
    if compute_deg:
        out_type.append(jax.ShapeDtypeStruct((_NC, _NACC), jnp.float32))
    scratch = [
        pltpu.VMEM((chunks, _K), jnp.int32),
        pltpu.VMEM((chunks, _K), jnp.int32),
        [pltpu.VMEM((_K, _DH), jnp.float32) for _ in range(2)],
        pltpu.VMEM((_K,), jnp.float32),
        pltpu.VMEM_SHARED((_NACC, _DH), jnp.float32),
        pltpu.VMEM_SHARED((_NACC,), jnp.float32),
        [pltpu.SemaphoreType.DMA for _ in range(2)],
        [pltpu.SemaphoreType.DMA for _ in range(2)],
    ]

    def body(*refs):
        if compute_deg:
            (table, srcs, dsts, out_acc, out_deg,
             src_i, dst_i, rows, ones_v, acc_sh, deg_sh, sem_g, sem_s) = refs
        else:
            (table, srcs, dsts, out_acc,
             src_i, dst_i, rows, ones_v, acc_sh, deg_sh, sem_g, sem_s) = refs
        c = lax.axis_index("c")
        s = lax.axis_index("s")
        base = s * _RPT

        zero16 = jnp.zeros((_L,), jnp.float32)

        def _zrow(i, carry):
            for jj in range(_DH // _L):
                rows[0][i, pl.ds(jj * _L, _L)] = zero16
            return carry

        lax.fori_loop(0, _K, _zrow, 0)
        for jj in range(_K // _L):
            ones_v[pl.ds(jj * _L, _L)] = jnp.full((_L,), 1.0, jnp.float32)
        for k in range(_RPT // _K):
            pltpu.sync_copy(rows[0], acc_sh.at[pl.ds(base + k * _K, _K)])
        if compute_deg:
            for k in range(_RPT // _DH):
                pltpu.sync_copy(rows[0].at[0],
                                deg_sh.at[pl.ds(base + k * _DH, _DH)])
        plsc.subcore_barrier()

        pltpu.sync_copy(srcs.at[c, s], src_i)
        pltpu.sync_copy(dsts.at[s], dst_i)

        def _retire(j, b):
            pltpu.make_async_copy(rows[b], acc_sh.at[dst_i.at[j]],
                                  sem_s[b]).wait()
            if compute_deg:
                pltpu.make_async_copy(ones_v, deg_sh.at[dst_i.at[j]],
                                      sem_s[b]).wait()

        def _slot(j, b, first=False, last=False):
            pltpu.make_async_copy(table.at[src_i.at[j]], rows[b],
                                  sem_g[b]).wait()
            pltpu.async_copy(rows[b], acc_sh.at[dst_i.at[j]], sem_s[b],
                             add=True)
            if compute_deg:
                pltpu.async_copy(ones_v, deg_sh.at[dst_i.at[j]], sem_s[b],
                                 add=True)
            if not first:
                _retire(j - 1, 1 - b)
            if not last:
                pltpu.async_copy(table.at[src_i.at[j + 1]], rows[1 - b],
                                 sem_g[1 - b])

        pltpu.async_copy(table.at[src_i.at[0]], rows[0], sem_g[0])
        _slot(0, 0, first=True)
        _slot(1, 1)

        def _pair(p, carry):
            _slot(2 * p, 0)
            _slot(2 * p + 1, 1)
            return carry

        lax.fori_loop(1, chunks // 2 - 1, _pair, 0)
        _slot(chunks - 2, 0)
        _slot(chunks - 1, 1, last=True)
        _retire(chunks - 1, 1)
        plsc.subcore_barrier()

        pltpu.sync_copy(acc_sh.at[pl.ds(base, _RPT)],
                        out_acc.at[c, pl.ds(base, _RPT)])
        if compute_deg:
            pltpu.sync_copy(deg_sh.at[pl.ds(base, _RPT)],
                            out_deg.at[c, pl.ds(base, _RPT)])

    return pl.kernel(
        body, out_type=tuple(out_type), mesh=mesh, scratch_types=scratch,
        compiler_params=pltpu.CompilerParams(use_tc_tiling_on_sc=False))


def _kan(a, w0, w1, wb):
    e0 = jnp.exp(-((a + 1.0) * 0.5) ** 2)
    e1 = jnp.exp(-((a - 1.0) * 0.5) ** 2)
    sl = a * lax.logistic(a)
    kw = dict(preferred_element_type=jnp.float32, precision=lax.Precision.HIGHEST)
    return jnp.dot(e0, w0, **kw) + jnp.dot(e1, w1, **kw) + jnp.dot(sl, wb, **kw)


def _layernorm(h):
    mu = jnp.mean(h, axis=-1, keepdims=True)
    cent = h - mu
    var = jnp.mean(cent * cent, axis=-1, keepdims=True)
    return cent * lax.rsqrt(var + 1e-5)


def _mean_from_parts(acc_ref, deg_ref):
    d = jnp.maximum(deg_ref[:, 0], 1.0)
    a = jnp.concatenate([acc_ref[0], acc_ref[1]], axis=-1)
    return a / d[:, None]


def _kan_ln_body(acc_ref, deg_ref, w0, w1, wb, o_ref):
    a = _mean_from_parts(acc_ref, deg_ref)
    hh = _layernorm(_kan(a, w0[...], w1[...], wb[...]))
    o_ref[0] = hh[:, :_DH]
    o_ref[1] = hh[:, _DH:]


def _kan_ln_head_body(acc_ref, deg_ref, w0, w1, wb, h0, h1, hb, o_ref):
    a = _mean_from_parts(acc_ref, deg_ref)
    h = _layernorm(_kan(a, w0[...], w1[...], wb[...]))
    o_ref[...] = _kan(h, h0[...], h1[...], hb[...])


_BLK = 400
_GRID = _N // _BLK


def _tc_specs(n_small):
    full = pl.BlockSpec((_D, _D), lambda i: (0, 0))
    small = pl.BlockSpec((_D, _T), lambda i: (0, 0))
    return ([pl.BlockSpec((_NC, _BLK, _DH), lambda i: (0, i, 0)),
             pl.BlockSpec((_BLK, _NC), lambda i: (i, 0))]
            + [full] * 3 + [small] * n_small)


def _stack_halves(t):
    return jnp.concatenate([t[:, :_DH], t[:, _DH:]], axis=0)


def kernel(x, edge_index, Ws0, Wb0, Ws1, Wb1, Hs, Hb):
    e = edge_index.shape[1]
    etot = e + _N
    chunks = -(-etot // (_NS * _K))
    chunks += chunks % 2
    epad = _NS * chunks * _K

    loop = jnp.arange(_N, dtype=jnp.int32)
    npad = epad - etot
    pad_dst = _N + jnp.arange(npad, dtype=jnp.int32) % (_NACC - _N)
    src = jnp.concatenate([
        edge_index[0].astype(jnp.int32), loop,
        jnp.zeros(npad, jnp.int32)]).reshape(_NS, chunks, _K)
    src = jnp.stack([src, src + _N])
    dst = jnp.concatenate([
        edge_index[1].astype(jnp.int32), loop,
        pad_dst]).reshape(_NS, chunks, _K)

    w00, w01, wb0 = Ws0[:, 0::2].T, Ws0[:, 1::2].T, Wb0.T
    w10, w11, wb1 = Ws1[:, 0::2].T, Ws1[:, 1::2].T, Wb1.T
    h0, h1, hb = Hs[:, 0::2].T, Hs[:, 1::2].T, Hb.T

    acc1, deg = _sc_segment_sum(chunks, True)(_stack_halves(x), src, dst)
    deg_t = deg.T

    h = pl.pallas_call(
        _kan_ln_body,
        grid=(_GRID,),
        in_specs=_tc_specs(0),
        out_specs=pl.BlockSpec((_NC, _BLK, _DH), lambda i: (0, i, 0)),
        out_shape=jax.ShapeDtypeStruct((_NC, _N, _DH), jnp.float32),
    )(acc1, deg_t, w00, w01, wb0)

    (acc2,) = _sc_segment_sum(chunks, False)(
        h.reshape(_NC * _N, _DH), src, dst)

    out = pl.pallas_call(
        _kan_ln_head_body,
        grid=(_GRID,),
        in_specs=_tc_specs(3),
        out_specs=pl.BlockSpec((_BLK, _T), lambda i: (i, 0)),
        out_shape=jax.ShapeDtypeStruct((_N, _T), jnp.float32),
    )(acc2, deg_t, w10, w11, wb1, h0, h1, hb)
    return out
